# baseline (speedup 1.0000x reference)
import functools

import jax
import jax.numpy as jnp
from jax import lax
from jax.experimental import pallas as pl
from jax.experimental.pallas import tpu as pltpu, tpu_sc as plsc

_K = 9
_PAD_K = 16
_NW = 32
_CHUNK = 128


def _lorentz_post(y, s):
    col = lax.broadcasted_iota(jnp.int32, y.shape, 1)
    first = y[:, 0:1]
    t = jax.nn.sigmoid(first) * jnp.exp(s) + 1.1
    yn = jnp.where(col == 0, 0.0, y)
    ss = jnp.sum(yn * yn, axis=1, keepdims=True)
    scale = jnp.sqrt((t * t - 1.0) / jnp.maximum(ss, 1e-8))
    return jnp.where(col == 0, t, y * scale)


def _ll_block(x, W, b, s):
    g = jax.nn.gelu(x)
    y = lax.dot_general(g, W, (((1,), (1,)), ((), ())),
                        preferred_element_type=jnp.float32) + b
    return _lorentz_post(y, s)


def _ffn_kernel(xcn_ref, w1_ref, b1_ref, s1_ref, w2_ref, b2_ref, s2_ref,
                o_ref):
    x = jnp.transpose(xcn_ref[0], (1, 0))
    h = _ll_block(x, w1_ref[...], b1_ref[...], s1_ref[0, 0])
    h = _ll_block(h, w2_ref[...], b2_ref[...], s2_ref[0, 0])
    o_ref[0] = h + x


def _sel_kernel(y1v_ref, f_ref, idx_ref):
    C = y1v_ref.shape[1]
    f = jnp.transpose(y1v_ref[0], (1, 0))
    N = f.shape[0]
    ones_col = jnp.ones((N, 1), jnp.float32)
    zeros_pad = jnp.zeros((N, 128 - C - 1), jnp.float32)
    f_aug = jnp.concatenate([f, ones_col, zeros_pad], axis=1)

    col = lax.broadcasted_iota(jnp.int32, f_aug.shape, 1)
    f_signed = jnp.where(col == 0, -f_aug,
                         jnp.where(col < C, f_aug, 0.0))
    inner = lax.dot_general(f_signed, f_aug, (((1,), (1,)), ((), ())),
                            preferred_element_type=jnp.float32)

    rowid = lax.broadcasted_iota(jnp.int32, inner.shape, 0)
    base = pl.program_id(0) * N
    neg_inf = jnp.float32(-jnp.inf)
    big = jnp.int32(2 ** 30)
    rows = []
    m_prev = None
    for _ in range(_K):
        if m_prev is None:
            m = jnp.max(inner, axis=0, keepdims=True)
        else:
            m = jnp.max(jnp.where(inner < m_prev, inner, neg_inf),
                        axis=0, keepdims=True)
        hit = inner == m
        idxv = jnp.min(jnp.where(hit, rowid, big), axis=0, keepdims=True)
        rows.append(idxv + base)
        m_prev = m
    pad = [rows[0]] * (_PAD_K - _K)
    idx_ref[0] = jnp.concatenate(rows + pad, axis=0)
    f_ref[0] = f_aug


def _sc_gather(table_hbm, idx_hbm, out_hbm, idx_v, rows_v, sem):
    wid = lax.axis_index("s") * 2 + lax.axis_index("c")
    rows_per_w = idx_hbm.shape[0] // _NW
    nchunk = rows_per_w // _CHUNK

    @pl.loop(0, nchunk)
    def body(i):
        start = wid * rows_per_w + i * _CHUNK
        pltpu.sync_copy(idx_hbm.at[pl.ds(start, _CHUNK)], idx_v)
        pltpu.async_copy(table_hbm.at[idx_v], rows_v, sem).wait()
        pltpu.sync_copy(rows_v, out_hbm.at[pl.ds(start, _CHUNK)])


def _tail_kernel(gath_ref, f_ref, scu_ref,
                 wg1_ref, wg2_ref, bg_ref, sg_ref,
                 w2a_ref, b2a_ref, s2a_ref, w2b_ref, b2b_ref, s2b_ref,
                 o_ref):
    C = wg1_ref.shape[0]
    f = f_ref[0][:, :C]
    nbmax = jnp.max(gath_ref[0], axis=0)[:, :C]
    rel = nbmax - f
    y = (lax.dot_general(jax.nn.gelu(f), wg1_ref[...],
                         (((1,), (1,)), ((), ())),
                         preferred_element_type=jnp.float32)
         + lax.dot_general(jax.nn.gelu(rel), wg2_ref[...],
                           (((1,), (1,)), ((), ())),
                           preferred_element_type=jnp.float32)
         + bg_ref[...])
    out = _lorentz_post(y, sg_ref[0, 0])

    h = _ll_block(out, w2a_ref[...], b2a_ref[...], s2a_ref[0, 0])
    h = _ll_block(h, w2b_ref[...], b2b_ref[...], s2b_ref[0, 0])
    o_ref[0] = h + out + scu_ref[0]


@functools.partial(jax.jit, static_argnames=())
def kernel(x, W1a, b1a, s1a, W1b, b1b, s1b, Wg, bg, sg, W2a, b2a, s2a,
           W2b, b2b, s2b):
    B, C, H, W = x.shape
    N = H * W
    f32 = jnp.float32

    def v(a):
        return jnp.asarray(a, f32).reshape(1, -1)

    def sc(a):
        return jnp.asarray(a, f32).reshape(1, 1)

    xcn = x.reshape(B, C, N)

    wspec = pl.BlockSpec((C, C), lambda b: (0, 0))
    bspec = pl.BlockSpec((1, C), lambda b: (0, 0))
    sspec = pl.BlockSpec((1, 1), lambda b: (0, 0))

    ffn1 = pl.pallas_call(
        _ffn_kernel,
        grid=(B,),
        in_specs=[
            pl.BlockSpec((1, C, N), lambda b: (b, 0, 0)),
            wspec, bspec, sspec, wspec, bspec, sspec,
        ],
        out_specs=pl.BlockSpec((1, N, C), lambda b: (b, 0, 0)),
        out_shape=jax.ShapeDtypeStruct((B, N, C), f32),
    )
    y1 = ffn1(xcn, W1a, v(b1a), sc(s1a), W1b, v(b1b), sc(s1b))

    y1v = y1.reshape(B, C, N)
    scu = x.reshape(B, N, C)

    sel = pl.pallas_call(
        _sel_kernel,
        grid=(B,),
        in_specs=[pl.BlockSpec((1, C, N), lambda b: (b, 0, 0))],
        out_specs=[
            pl.BlockSpec((1, N, 128), lambda b: (b, 0, 0)),
            pl.BlockSpec((1, _PAD_K, N), lambda b: (b, 0, 0)),
        ],
        out_shape=[
            jax.ShapeDtypeStruct((B, N, 128), f32),
            jax.ShapeDtypeStruct((B, _PAD_K, N), jnp.int32),
        ],
    )
    f_all, idxg = sel(y1v)

    table = f_all.reshape(B * N, 128)
    idx_flat = idxg[:, :_K, :].reshape(B * _K * N)

    gather = pl.kernel(
        _sc_gather,
        out_type=jax.ShapeDtypeStruct((B * _K * N, 128), f32),
        mesh=plsc.VectorSubcoreMesh(core_axis_name="c", subcore_axis_name="s"),
        scratch_types=[
            pltpu.VMEM((_CHUNK,), jnp.int32),
            pltpu.VMEM((_CHUNK, 128), f32),
            pltpu.SemaphoreType.DMA,
        ],
    )
    gath = gather(table, idx_flat).reshape(B, _K, N, 128)

    Wg1 = Wg[:, :C]
    Wg2 = Wg[:, C:]

    tail = pl.pallas_call(
        _tail_kernel,
        grid=(B,),
        in_specs=[
            pl.BlockSpec((1, _K, N, 128), lambda b: (b, 0, 0, 0)),
            pl.BlockSpec((1, N, 128), lambda b: (b, 0, 0)),
            pl.BlockSpec((1, N, C), lambda b: (b, 0, 0)),
            wspec, wspec, bspec, sspec,
            wspec, bspec, sspec, wspec, bspec, sspec,
        ],
        out_specs=pl.BlockSpec((1, N, C), lambda b: (b, 0, 0)),
        out_shape=jax.ShapeDtypeStruct((B, N, C), f32),
    )
    z = tail(gath, f_all, scu, Wg1, Wg2, v(bg), sc(sg),
             W2a, v(b2a), sc(s2a), W2b, v(b2b), sc(s2b))

    return z.reshape(B, C, H, W)


# speedup vs baseline: 1.0287x; 1.0287x over previous
import functools

import jax
import jax.numpy as jnp
from jax import lax
from jax.experimental import pallas as pl
from jax.experimental.pallas import tpu as pltpu, tpu_sc as plsc

_K = 9
_PAD_K = 16
_NW = 32
_CHUNK = 128


def _lorentz_post(y, s):
    col = lax.broadcasted_iota(jnp.int32, y.shape, 1)
    first = y[:, 0:1]
    t = jax.nn.sigmoid(first) * jnp.exp(s) + 1.1
    yn = jnp.where(col == 0, 0.0, y)
    ss = jnp.sum(yn * yn, axis=1, keepdims=True)
    scale = jnp.sqrt((t * t - 1.0) / jnp.maximum(ss, 1e-8))
    return jnp.where(col == 0, t, y * scale)


def _ll_block(x, W, b, s):
    g = jax.nn.gelu(x)
    y = lax.dot_general(g, W, (((1,), (1,)), ((), ())),
                        preferred_element_type=jnp.float32) + b
    return _lorentz_post(y, s)


def _ffn_kernel(xcn_ref, w1_ref, b1_ref, s1_ref, w2_ref, b2_ref, s2_ref,
                o_ref):
    x = jnp.transpose(xcn_ref[0], (1, 0))
    h = _ll_block(x, w1_ref[...], b1_ref[...], s1_ref[0, 0])
    h = _ll_block(h, w2_ref[...], b2_ref[...], s2_ref[0, 0])
    o_ref[0] = h + x


def _sel_kernel(y1v_ref, f_ref, idx_ref):
    C = y1v_ref.shape[1]
    f = jnp.transpose(y1v_ref[0], (1, 0))
    N = f.shape[0]
    ones_col = jnp.ones((N, 1), jnp.float32)
    zeros_pad = jnp.zeros((N, 128 - C - 1), jnp.float32)
    f_aug = jnp.concatenate([f, ones_col, zeros_pad], axis=1)

    col = lax.broadcasted_iota(jnp.int32, f_aug.shape, 1)
    f_signed = jnp.where(col == 0, -f_aug,
                         jnp.where(col < C, f_aug, 0.0))
    inner = lax.dot_general(f_signed, f_aug, (((1,), (1,)), ((), ())),
                            preferred_element_type=jnp.float32)

    rowid = lax.broadcasted_iota(jnp.int32, inner.shape, 0)
    base = pl.program_id(0) * N
    neg_inf = jnp.float32(-jnp.inf)
    big = jnp.int32(2 ** 30)
    rows = []
    m_prev = None
    for _ in range(_K):
        if m_prev is None:
            m = jnp.max(inner, axis=0, keepdims=True)
        else:
            m = jnp.max(jnp.where(inner < m_prev, inner, neg_inf),
                        axis=0, keepdims=True)
        hit = inner == m
        idxv = jnp.min(jnp.where(hit, rowid, big), axis=0, keepdims=True)
        rows.append(idxv + base)
        m_prev = m
    pad = [rows[0]] * (_PAD_K - _K)
    idx_ref[0] = jnp.concatenate(rows + pad, axis=0)
    f_ref[0] = f_aug


def _sc_gather(table_hbm, idx_hbm, out_hbm, idx_all, rows_a, rows_b,
               sem_a, sem_b):
    wid = lax.axis_index("s") * 2 + lax.axis_index("c")
    rows_per_w = idx_hbm.shape[0] // _NW
    base = wid * rows_per_w
    pltpu.sync_copy(idx_hbm.at[pl.ds(base, rows_per_w)], idx_all)
    nchunk = rows_per_w // _CHUNK
    bufs = (rows_a, rows_b)
    sems = (sem_a, sem_b)

    @pl.loop(0, nchunk, step=2)
    def body(i):
        cps = []
        for b in range(2):
            off = (i + b) * _CHUNK
            cps.append((off, pltpu.async_copy(
                table_hbm.at[idx_all.at[pl.ds(off, _CHUNK)]],
                bufs[b], sems[b])))
        for b in range(2):
            off, cp = cps[b]
            cp.wait()
            pltpu.sync_copy(bufs[b], out_hbm.at[pl.ds(base + off, _CHUNK)])


def _tail_kernel(gath_ref, f_ref, scu_ref,
                 wg1_ref, wg2_ref, bg_ref, sg_ref,
                 w2a_ref, b2a_ref, s2a_ref, w2b_ref, b2b_ref, s2b_ref,
                 o_ref):
    C = wg1_ref.shape[0]
    f = f_ref[0][:, :C]
    nbmax = jnp.max(gath_ref[0], axis=0)[:, :C]
    rel = nbmax - f
    y = (lax.dot_general(jax.nn.gelu(f), wg1_ref[...],
                         (((1,), (1,)), ((), ())),
                         preferred_element_type=jnp.float32)
         + lax.dot_general(jax.nn.gelu(rel), wg2_ref[...],
                           (((1,), (1,)), ((), ())),
                           preferred_element_type=jnp.float32)
         + bg_ref[...])
    out = _lorentz_post(y, sg_ref[0, 0])

    h = _ll_block(out, w2a_ref[...], b2a_ref[...], s2a_ref[0, 0])
    h = _ll_block(h, w2b_ref[...], b2b_ref[...], s2b_ref[0, 0])
    o_ref[0] = h + out + scu_ref[0]


@functools.partial(jax.jit, static_argnames=())
def kernel(x, W1a, b1a, s1a, W1b, b1b, s1b, Wg, bg, sg, W2a, b2a, s2a,
           W2b, b2b, s2b):
    B, C, H, W = x.shape
    N = H * W
    f32 = jnp.float32

    def v(a):
        return jnp.asarray(a, f32).reshape(1, -1)

    def sc(a):
        return jnp.asarray(a, f32).reshape(1, 1)

    xcn = x.reshape(B, C, N)

    wspec = pl.BlockSpec((C, C), lambda b: (0, 0))
    bspec = pl.BlockSpec((1, C), lambda b: (0, 0))
    sspec = pl.BlockSpec((1, 1), lambda b: (0, 0))

    ffn1 = pl.pallas_call(
        _ffn_kernel,
        grid=(B,),
        in_specs=[
            pl.BlockSpec((1, C, N), lambda b: (b, 0, 0)),
            wspec, bspec, sspec, wspec, bspec, sspec,
        ],
        out_specs=pl.BlockSpec((1, N, C), lambda b: (b, 0, 0)),
        out_shape=jax.ShapeDtypeStruct((B, N, C), f32),
    )
    y1 = ffn1(xcn, W1a, v(b1a), sc(s1a), W1b, v(b1b), sc(s1b))

    y1v = y1.reshape(B, C, N)
    scu = x.reshape(B, N, C)

    sel = pl.pallas_call(
        _sel_kernel,
        grid=(B,),
        in_specs=[pl.BlockSpec((1, C, N), lambda b: (b, 0, 0))],
        out_specs=[
            pl.BlockSpec((1, N, 128), lambda b: (b, 0, 0)),
            pl.BlockSpec((1, _PAD_K, N), lambda b: (b, 0, 0)),
        ],
        out_shape=[
            jax.ShapeDtypeStruct((B, N, 128), f32),
            jax.ShapeDtypeStruct((B, _PAD_K, N), jnp.int32),
        ],
    )
    f_all, idxg = sel(y1v)

    table = f_all.reshape(B * N, 128)
    idx_flat = idxg[:, :_K, :].reshape(B * _K * N)

    gather = pl.kernel(
        _sc_gather,
        out_type=jax.ShapeDtypeStruct((B * _K * N, 128), f32),
        mesh=plsc.VectorSubcoreMesh(core_axis_name="c", subcore_axis_name="s"),
        scratch_types=[
            pltpu.VMEM((B * _K * N // _NW,), jnp.int32),
            pltpu.VMEM((_CHUNK, 128), f32),
            pltpu.VMEM((_CHUNK, 128), f32),
            pltpu.SemaphoreType.DMA,
            pltpu.SemaphoreType.DMA,
        ],
    )
    gath = gather(table, idx_flat).reshape(B, _K, N, 128)

    Wg1 = Wg[:, :C]
    Wg2 = Wg[:, C:]

    tail = pl.pallas_call(
        _tail_kernel,
        grid=(B,),
        in_specs=[
            pl.BlockSpec((1, _K, N, 128), lambda b: (b, 0, 0, 0)),
            pl.BlockSpec((1, N, 128), lambda b: (b, 0, 0)),
            pl.BlockSpec((1, N, C), lambda b: (b, 0, 0)),
            wspec, wspec, bspec, sspec,
            wspec, bspec, sspec, wspec, bspec, sspec,
        ],
        out_specs=pl.BlockSpec((1, N, C), lambda b: (b, 0, 0)),
        out_shape=jax.ShapeDtypeStruct((B, N, C), f32),
    )
    z = tail(gath, f_all, scu, Wg1, Wg2, v(bg), sc(sg),
             W2a, v(b2a), sc(s2a), W2b, v(b2b), sc(s2b))

    return z.reshape(B, C, H, W)
